# baseline (speedup 1.0000x reference)
import functools

import jax
import jax.numpy as jnp
from jax import lax
from jax.experimental import pallas as pl
from jax.experimental.pallas import tpu as pltpu
from jax.experimental.pallas import tpu_sc as plsc

N_NODES = 100000
H = 16
G = 512
L = 7
N_CH = 782
N_P = N_CH * 128
N_PAD = N_P - N_NODES
E_EDGES = 3200000
NW = 32
E_P = 3212288
E_CH = E_P // 128
NODES_W = N_P // NW
GR_W = G // NW
WIN_CH = 783

_mesh = plsc.VectorSubcoreMesh(core_axis_name="c", subcore_axis_name="s")
_sc_params = pltpu.CompilerParams(use_tc_tiling_on_sc=False)


def _seq_scatter_rows(idx_row, rows, acc, base, lo, hi, cbase, nrows):
    for g in range(128 // 16):
        pos = cbase + g * 16 + lax.iota(jnp.int32, 16)
        dvec = idx_row[0, pl.ds(g * 16, 16)] - base
        ok = (pos >= lo) & (pos < hi)
        dvec = jnp.where(ok, dvec, nrows)
        for j in range(16):
            dl = dvec[j]
            acc[dl, :] = acc[dl, :] + rows[g * 16 + j, :]


@functools.partial(
    pl.kernel,
    mesh=_mesh,
    out_type=(
        jax.ShapeDtypeStruct((N_P, H), jnp.float32),
        jax.ShapeDtypeStruct((G, H), jnp.float32),
    ),
    scratch_types=[
        pltpu.VMEM((1, 16), jnp.int32),
        pltpu.VMEM((1, 16), jnp.int32),
        pltpu.VMEM((1, 128), jnp.int32),
        pltpu.VMEM((1, 128), jnp.int32),
        pltpu.VMEM((128, H), jnp.float32),
        pltpu.VMEM((128, H), jnp.float32),
        pltpu.VMEM((NODES_W + 8, H), jnp.float32),
        pltpu.VMEM((NODES_W + 8, H), jnp.float32),
        pltpu.VMEM((GR_W + 8, H), jnp.float32),
        pltpu.SemaphoreType.DMA,
    ],
    compiler_params=_sc_params,
)
def _sc_agg_pool(h_hbm, src_hbm, dst_hbm, batch_hbm, eb_hbm, nb_hbm,
                 agg_out, pool_out,
                 ebv, nbv, sidx, didx, rows, prow, acc, acc2, pacc, sem):
    cid = lax.axis_index("c")
    sid = lax.axis_index("s")
    wid = cid * 16 + sid

    pltpu.sync_copy(eb_hbm.at[pl.ds(wid, 1)], ebv)
    pltpu.sync_copy(nb_hbm.at[pl.ds(wid, 1)], nbv)
    eb = ebv[0, 0:16]
    nb = nbv[0, 0:16]
    e_lo, e_hi = eb[0], eb[1]
    n_lo, n_hi = nb[0], nb[1]

    zrow = jnp.zeros((H,), jnp.float32)

    @pl.loop(0, NODES_W + 8)
    def _(i):
        acc[i, :] = zrow
        acc2[i, :] = zrow

    @pl.loop(0, GR_W + 8)
    def _(i):
        pacc[i, :] = zrow

    nbase = wid * NODES_W

    @pl.loop(e_lo // 128, (e_hi + 127) // 128)
    def _(c):
        @pl.when((c % WIN_CH == 0) & (c > 0))
        def _():
            @pl.loop(0, NODES_W)
            def _(i):
                acc2[i, :] = acc2[i, :] + acc[i, :]
                acc[i, :] = zrow

        pltpu.sync_copy(src_hbm.at[pl.ds(c, 1)], sidx)
        pltpu.sync_copy(dst_hbm.at[pl.ds(c, 1)], didx)
        pltpu.async_copy(h_hbm.at[sidx.at[0]], rows, sem).wait()
        _seq_scatter_rows(didx, rows, acc, nbase, e_lo, e_hi, c * 128, NODES_W)

    @pl.loop(0, NODES_W)
    def _(i):
        acc2[i, :] = acc2[i, :] + acc[i, :]

    gbase = wid * GR_W

    @pl.loop(n_lo // 128, (n_hi + 127) // 128)
    def _(c):
        pltpu.sync_copy(batch_hbm.at[pl.ds(c, 1)], didx)
        pltpu.sync_copy(h_hbm.at[pl.ds(c * 128, 128)], prow)
        _seq_scatter_rows(didx, prow, pacc, gbase, n_lo, n_hi, c * 128, GR_W)

    pltpu.sync_copy(acc2.at[pl.ds(0, NODES_W)],
                    agg_out.at[pl.ds(wid * NODES_W, NODES_W)])
    pltpu.sync_copy(pacc.at[pl.ds(0, GR_W)],
                    pool_out.at[pl.ds(wid * GR_W, GR_W)])


@functools.partial(
    pl.kernel,
    mesh=_mesh,
    out_type=jax.ShapeDtypeStruct((G, H), jnp.float32),
    scratch_types=[
        pltpu.VMEM((1, 16), jnp.int32),
        pltpu.VMEM((1, 128), jnp.int32),
        pltpu.VMEM((128, H), jnp.float32),
        pltpu.VMEM((GR_W + 8, H), jnp.float32),
    ],
    compiler_params=_sc_params,
)
def _sc_pool(h_hbm, batch_hbm, nb_hbm, pool_out, nbv, bidx, prow, pacc):
    cid = lax.axis_index("c")
    sid = lax.axis_index("s")
    wid = cid * 16 + sid

    pltpu.sync_copy(nb_hbm.at[pl.ds(wid, 1)], nbv)
    nb = nbv[0, 0:16]
    n_lo, n_hi = nb[0], nb[1]

    zrow = jnp.zeros((H,), jnp.float32)

    @pl.loop(0, GR_W + 8)
    def _(i):
        pacc[i, :] = zrow

    gbase = wid * GR_W

    @pl.loop(n_lo // 128, (n_hi + 127) // 128)
    def _(c):
        pltpu.sync_copy(batch_hbm.at[pl.ds(c, 1)], bidx)
        pltpu.sync_copy(h_hbm.at[pl.ds(c * 128, 128)], prow)
        _seq_scatter_rows(bidx, prow, pacc, gbase, n_lo, n_hi, c * 128, GR_W)

    pltpu.sync_copy(pacc.at[pl.ds(0, GR_W)],
                    pool_out.at[pl.ds(wid * GR_W, GR_W)])


N_ROWS = N_P // 8
N_REAL = N_NODES // 8


def _tc_layer_body(h_ref, agg_ref, w1_ref, b1t_ref, b1s_ref, g_ref, bet_ref,
                   w2_ref, b2t_ref, o_ref):
    h = h_ref[...]
    mask = lax.broadcasted_iota(jnp.int32, (N_ROWS, 1), 0) < N_REAL
    z = h + agg_ref[...]
    z = jnp.dot(z.astype(jnp.bfloat16), w1_ref[...].astype(jnp.bfloat16),
                preferred_element_type=jnp.float32) + b1t_ref[...]
    b1s = b1s_ref[...]

    def _fold(v):
        acc = v[:, 0:H]
        for i in range(1, 8):
            acc = acc + v[:, i * H:(i + 1) * H]
        return acc

    def _spread(v):
        return jnp.concatenate([v] * 8, axis=1)

    s = _fold(jnp.sum(z, axis=0, keepdims=True))
    mean16 = (s - N_PAD * b1s) / N_NODES
    d = z - _spread(mean16)
    ssq = _fold(jnp.sum(d * d, axis=0, keepdims=True))
    dp = b1s - mean16
    var16 = (ssq - N_PAD * dp * dp) / N_NODES
    zn = d / _spread(jnp.sqrt(var16 + 1e-5)) * _spread(g_ref[...])
    zn = zn + bet_ref[...]
    a = jnp.maximum(zn, 0.0)
    z2 = jnp.dot(a.astype(jnp.bfloat16), w2_ref[...].astype(jnp.bfloat16),
                 preferred_element_type=jnp.float32) + b2t_ref[...]
    o_ref[...] = jnp.where(mask, jnp.maximum(z2, 0.0), 0.0)


def _tc_layer(h2, aggp2, w1b, b1t, b1s, gamma, betat, w2b, b2t):
    return pl.pallas_call(
        _tc_layer_body,
        out_shape=jax.ShapeDtypeStruct((N_ROWS, 128), jnp.float32),
    )(h2, aggp2, w1b, b1t, b1s, gamma, betat, w2b, b2t)


def _tc_readout_body(p_ref, w1_ref, b1_ref, w2_ref, b2_ref, sig_ref, out_ref):
    p = p_ref[...]
    hcat = jnp.concatenate([p[i] for i in range(L)], axis=1)
    h1 = jnp.dot(hcat.astype(jnp.bfloat16), w1_ref[...].astype(jnp.bfloat16),
                 preferred_element_type=jnp.float32)
    h1 = jnp.maximum(h1 + b1_ref[...], 0.0)
    out = jnp.sum(h1.astype(jnp.bfloat16).astype(jnp.float32)
                  * w2_ref[...].astype(jnp.bfloat16).astype(jnp.float32),
                  axis=1, keepdims=True) + b2_ref[...]
    out_ref[...] = out
    sig_ref[...] = jax.nn.sigmoid(out)


def _tc_readout(pools, lin1_w, lin1_b, lin2_w, lin2_b):
    return pl.pallas_call(
        _tc_readout_body,
        out_shape=(jax.ShapeDtypeStruct((G, 1), jnp.float32),
                   jax.ShapeDtypeStruct((G, 1), jnp.float32)),
    )(pools, lin1_w, lin1_b, lin2_w, lin2_b)


def kernel(x, edge_index, batch, W1, b1, gamma, beta, W2, b2,
           lin1_w, lin1_b, lin2_w, lin2_b):
    src = edge_index[0]
    dst = edge_index[1]
    dst_s, order = lax.sort_key_val(dst, lax.iota(jnp.int32, E_EDGES),
                                    is_stable=True)
    src_s = jnp.take(src, order, axis=0)
    epad = E_P - E_EDGES
    src_p = jnp.concatenate(
        [src_s, jnp.zeros((epad,), jnp.int32)]).reshape(E_CH, 128)
    dst_p = jnp.concatenate(
        [dst_s, jnp.full((epad,), N_NODES, jnp.int32)]).reshape(E_CH, 128)
    batch_p = jnp.concatenate(
        [batch, jnp.zeros((N_PAD,), jnp.int32)]).reshape(N_CH, 128)
    h = jnp.pad(x, ((0, N_PAD), (0, 0)))

    eb = jnp.searchsorted(dst_s, jnp.arange(NW + 1, dtype=jnp.int32) * NODES_W)
    nb = jnp.searchsorted(batch, jnp.arange(NW + 1, dtype=jnp.int32) * GR_W)
    ebw = jnp.zeros((NW, 16), jnp.int32).at[:, 0].set(eb[:-1]).at[:, 1].set(eb[1:])
    nbw = jnp.zeros((NW, 16), jnp.int32).at[:, 0].set(nb[:-1]).at[:, 1].set(nb[1:])
    ebw = ebw.astype(jnp.int32)
    nbw = nbw.astype(jnp.int32)

    eye8 = jnp.eye(8, dtype=jnp.float32)
    w1b = jnp.einsum("ab,lij->laibj", eye8, W1).reshape(L, 128, 128)
    w2b = jnp.einsum("ab,lij->laibj", eye8, W2).reshape(L, 128, 128)
    b1t = jnp.tile(b1, (1, 8))
    b2t = jnp.tile(b2, (1, 8))
    betat = jnp.tile(beta, (1, 8))

    pools = []
    for i in range(L):
        aggp, poolp = _sc_agg_pool(h, src_p, dst_p, batch_p, ebw, nbw)
        if i > 0:
            pools.append(poolp)
        h2 = _tc_layer(h.reshape(N_ROWS, 128), aggp.reshape(N_ROWS, 128),
                       w1b[i], b1t[i].reshape(1, 128), b1[i].reshape(1, H),
                       gamma[i].reshape(1, H), betat[i].reshape(1, 128),
                       w2b[i], b2t[i].reshape(1, 128))
        h = h2.reshape(N_P, H)
    pools.append(_sc_pool(h, batch_p, nbw))
    allp = jnp.stack(pools)
    sig, out = _tc_readout(allp, lin1_w, lin1_b.reshape(1, L * H),
                           lin2_w.reshape(1, L * H), lin2_b.reshape(1, 1))
    return (sig, out)
